# baseline (speedup 1.0000x reference)
import functools

import jax
import jax.numpy as jnp
from jax import lax
from jax.experimental import pallas as pl
from jax.experimental.pallas import tpu as pltpu
from jax.experimental.pallas import tpu_sc as plsc

B = 4096
L = 20
D = 32
H1 = 256
OUT = 64

NW = 32
BPW = B // NW
GPW = BPW * L
GCH = 128
NCH = GPW // GCH

_MESH = plsc.VectorSubcoreMesh(core_axis_name="c", subcore_axis_name="s")


def _sc_genre_sum(genre_ids_flat, genre_table):
    @functools.partial(
        pl.kernel,
        mesh=_MESH,
        out_type=jax.ShapeDtypeStruct((B, D), jnp.float32),
        scratch_types=[
            pltpu.VMEM((GPW,), jnp.int32),
            pltpu.VMEM((GPW, D), jnp.float32),
            pltpu.VMEM((BPW, D), jnp.float32),
            pltpu.SemaphoreType.DMA,
        ],
        compiler_params=pltpu.CompilerParams(use_tc_tiling_on_sc=False),
    )
    def genre_kernel(gids_hbm, gtab_hbm, out_hbm, gidx, grows, gsum, sem):
        wid = lax.axis_index("s") * 2 + lax.axis_index("c")
        base = wid * BPW
        pltpu.sync_copy(gids_hbm.at[pl.ds(base * L, GPW)], gidx)
        copies = []
        for j in range(NCH):
            copies.append(
                pltpu.async_copy(
                    gtab_hbm.at[gidx.at[pl.ds(j * GCH, GCH)]],
                    grows.at[pl.ds(j * GCH, GCH)],
                    sem,
                )
            )
        for cp in copies:
            cp.wait()

        def body(b, carry):
            for j in range(D // 16):
                acc = grows[b * L, pl.ds(j * 16, 16)]
                for l in range(1, L):
                    acc = acc + grows[b * L + l, pl.ds(j * 16, 16)]
                gsum[b, pl.ds(j * 16, 16)] = acc
            return carry

        lax.fori_loop(0, BPW, body, 0)
        pltpu.sync_copy(gsum, out_hbm.at[pl.ds(base, BPW)])

    return genre_kernel(genre_ids_flat, genre_table)


def _sc_movie_rows(movie_ids, movie_table):
    @functools.partial(
        pl.kernel,
        mesh=_MESH,
        out_type=jax.ShapeDtypeStruct((B, D), jnp.float32),
        scratch_types=[
            pltpu.VMEM((BPW,), jnp.int32),
            pltpu.VMEM((BPW, D), jnp.float32),
            pltpu.SemaphoreType.DMA,
        ],
        compiler_params=pltpu.CompilerParams(use_tc_tiling_on_sc=False),
    )
    def movie_kernel(mids_hbm, mtab_hbm, out_hbm, midx, mrows, sem):
        wid = lax.axis_index("s") * 2 + lax.axis_index("c")
        base = wid * BPW
        pltpu.sync_copy(mids_hbm.at[pl.ds(base, BPW)], midx)
        pltpu.async_copy(mtab_hbm.at[midx], mrows, sem).wait()
        pltpu.sync_copy(mrows, out_hbm.at[pl.ds(base, BPW)])

    return movie_kernel(movie_ids, movie_table)


def _tc_pack(table_t, vocab_pad):
    BW = 1024
    grid = (vocab_pad // BW,)

    def pack_body(in_ref, out_ref):
        r_iota = lax.broadcasted_iota(jnp.int32, (128, 128), 0)
        c_iota = lax.broadcasted_iota(jnp.int32, (128, 128), 1)
        sel = (c_iota == 4 * (r_iota % D) + r_iota // D).astype(jnp.float32)
        x = in_ref[...]
        for t in range(BW // 128):
            xt = x[:, t * 128:(t + 1) * 128]
            y = lax.dot_general(sel, xt, (((1,), (1,)), ((), ())),
                                preferred_element_type=jnp.float32)
            out_ref[t * D:(t + 1) * D, :] = jnp.concatenate(
                [y[q * D:(q + 1) * D, :] for q in range(4)], axis=1)

    return pl.pallas_call(
        pack_body,
        grid=grid,
        in_specs=[pl.BlockSpec((D, BW), lambda k: (0, k))],
        out_specs=pl.BlockSpec((D * BW // 128, 128), lambda k: (k, 0)),
        out_shape=jax.ShapeDtypeStruct((vocab_pad * D // 128, 128),
                                       jnp.float32),
        compiler_params=pltpu.CompilerParams(
            dimension_semantics=("arbitrary",),
        ),
    )(table_t)


def _tc_mlp(m, gsum, genre_ids, row0, W1, b1, W2, b2):
    BLK = 512

    def mlp_body(m_ref, g_ref, gid_ref, row0_ref, W1_ref, b1_ref, W2_ref,
                 b2_ref, out_ref):
        gids = gid_ref[...]
        c0 = jnp.sum((gids == 0).astype(jnp.float32), axis=1, keepdims=True)
        denom = jnp.maximum(jnp.float32(L) - c0, 1.0)
        g = (g_ref[...] - c0 * row0_ref[...]) / denom
        h = jnp.maximum(
            jnp.dot(m_ref[...], W1_ref[:D, :], preferred_element_type=jnp.float32)
            + jnp.dot(g, W1_ref[D:, :], preferred_element_type=jnp.float32)
            + b1_ref[...],
            0.0,
        )
        out_ref[...] = (
            jnp.dot(h, W2_ref[...], preferred_element_type=jnp.float32)
            + b2_ref[...]
        )

    return pl.pallas_call(
        mlp_body,
        grid=(B // BLK,),
        in_specs=[
            pl.BlockSpec((BLK, D), lambda i: (i, 0)),
            pl.BlockSpec((BLK, D), lambda i: (i, 0)),
            pl.BlockSpec((BLK, L), lambda i: (i, 0)),
            pl.BlockSpec((1, D), lambda i: (0, 0)),
            pl.BlockSpec((2 * D, H1), lambda i: (0, 0)),
            pl.BlockSpec((1, H1), lambda i: (0, 0)),
            pl.BlockSpec((H1, OUT), lambda i: (0, 0)),
            pl.BlockSpec((1, OUT), lambda i: (0, 0)),
        ],
        out_specs=pl.BlockSpec((BLK, OUT), lambda i: (i, 0)),
        out_shape=jax.ShapeDtypeStruct((B, OUT), jnp.float32),
        compiler_params=pltpu.CompilerParams(
            dimension_semantics=("parallel",),
        ),
    )(m, gsum, genre_ids, row0, W1, b1, W2, b2)


def kernel(movie_ids, genre_ids, movie_table, genre_table, W1, b1, W2, b2):
    mids = movie_ids.astype(jnp.int32)
    gids = genre_ids.astype(jnp.int32)
    gt_lin = _tc_pack(genre_table.T, 10240).reshape(10240, D)
    mt_lin = _tc_pack(movie_table.T, 100352).reshape(100352, D)
    gsum = _sc_genre_sum(gids.reshape(B * L), gt_lin)
    m = _sc_movie_rows(mids, mt_lin)
    row0 = genre_table[0:1, :]
    return _tc_mlp(m, gsum, gids, row0,
                   W1, b1.reshape(1, H1), W2, b2.reshape(1, OUT))


# speedup vs baseline: 1.0147x; 1.0147x over previous
import functools

import jax
import jax.numpy as jnp
from jax import lax
from jax.experimental import pallas as pl
from jax.experimental.pallas import tpu as pltpu
from jax.experimental.pallas import tpu_sc as plsc

B = 4096
L = 20
D = 32
H1 = 256
OUT = 64

NW = 32
BPW = B // NW
GPW = BPW * L
GCH = 128
NCH = GPW // GCH

_MESH = plsc.VectorSubcoreMesh(core_axis_name="c", subcore_axis_name="s")


def _sc_genre_sum(genre_ids_flat, genre_table):
    @functools.partial(
        pl.kernel,
        mesh=_MESH,
        out_type=jax.ShapeDtypeStruct((B, D), jnp.float32),
        scratch_types=[
            pltpu.VMEM((GPW,), jnp.int32),
            pltpu.VMEM((GPW, D), jnp.float32),
            pltpu.VMEM((BPW, D), jnp.float32),
            pltpu.SemaphoreType.DMA,
        ],
        compiler_params=pltpu.CompilerParams(use_tc_tiling_on_sc=False),
    )
    def genre_kernel(gids_hbm, gtab_hbm, out_hbm, gidx, grows, gsum, sem):
        wid = lax.axis_index("s") * 2 + lax.axis_index("c")
        base = wid * BPW
        pltpu.sync_copy(gids_hbm.at[pl.ds(base * L, GPW)], gidx)
        copies = []
        for j in range(NCH):
            copies.append(
                pltpu.async_copy(
                    gtab_hbm.at[gidx.at[pl.ds(j * GCH, GCH)]],
                    grows.at[pl.ds(j * GCH, GCH)],
                    sem,
                )
            )
        for cp in copies:
            cp.wait()

        def body(b, carry):
            for j in range(D // 16):
                acc = grows[b * L, pl.ds(j * 16, 16)]
                for l in range(1, L):
                    acc = acc + grows[b * L + l, pl.ds(j * 16, 16)]
                gsum[b, pl.ds(j * 16, 16)] = acc
            return carry

        lax.fori_loop(0, BPW, body, 0)
        pltpu.sync_copy(gsum, out_hbm.at[pl.ds(base, BPW)])

    return genre_kernel(genre_ids_flat, genre_table)


def _sc_movie_rows(movie_ids, movie_table):
    @functools.partial(
        pl.kernel,
        mesh=_MESH,
        out_type=jax.ShapeDtypeStruct((B, D), jnp.float32),
        scratch_types=[
            pltpu.VMEM((BPW,), jnp.int32),
            pltpu.VMEM((BPW, D), jnp.float32),
            pltpu.SemaphoreType.DMA,
        ],
        compiler_params=pltpu.CompilerParams(use_tc_tiling_on_sc=False),
    )
    def movie_kernel(mids_hbm, mtab_hbm, out_hbm, midx, mrows, sem):
        wid = lax.axis_index("s") * 2 + lax.axis_index("c")
        base = wid * BPW
        pltpu.sync_copy(mids_hbm.at[pl.ds(base, BPW)], midx)
        pltpu.async_copy(mtab_hbm.at[midx], mrows, sem).wait()
        pltpu.sync_copy(mrows, out_hbm.at[pl.ds(base, BPW)])

    return movie_kernel(movie_ids, movie_table)


def _tc_pack(table_t, vocab_pad):
    BW = 1024
    grid = (vocab_pad // BW,)
    r_iota = jnp.arange(128, dtype=jnp.int32)[:, None]
    c_iota = jnp.arange(128, dtype=jnp.int32)[None, :]
    sel = (c_iota == 4 * (r_iota % D) + r_iota // D).astype(jnp.float32)

    def pack_body(sel_ref, in_ref, out_ref):
        s = sel_ref[...]
        x = in_ref[...]
        for t in range(BW // 128):
            xt = x[:, t * 128:(t + 1) * 128]
            y = lax.dot_general(s, xt, (((1,), (1,)), ((), ())),
                                preferred_element_type=jnp.float32)
            out_ref[t * D:(t + 1) * D, :] = jnp.concatenate(
                [y[q * D:(q + 1) * D, :] for q in range(4)], axis=1)

    return pl.pallas_call(
        pack_body,
        grid=grid,
        in_specs=[pl.BlockSpec((128, 128), lambda k: (0, 0)),
                  pl.BlockSpec((D, BW), lambda k: (0, k))],
        out_specs=pl.BlockSpec((D * BW // 128, 128), lambda k: (k, 0)),
        out_shape=jax.ShapeDtypeStruct((vocab_pad * D // 128, 128),
                                       jnp.float32),
        compiler_params=pltpu.CompilerParams(
            dimension_semantics=("arbitrary",),
        ),
    )(sel, table_t)


def _tc_mlp(m, gsum, genre_ids, row0, W1, b1, W2, b2):
    BLK = 1024

    def mlp_body(m_ref, g_ref, gid_ref, row0_ref, W1_ref, b1_ref, W2_ref,
                 b2_ref, out_ref):
        gids = gid_ref[...]
        c0 = jnp.sum((gids == 0).astype(jnp.float32), axis=1, keepdims=True)
        denom = jnp.maximum(jnp.float32(L) - c0, 1.0)
        g = (g_ref[...] - c0 * row0_ref[...]) / denom
        h = jnp.maximum(
            jnp.dot(m_ref[...], W1_ref[:D, :], preferred_element_type=jnp.float32)
            + jnp.dot(g, W1_ref[D:, :], preferred_element_type=jnp.float32)
            + b1_ref[...],
            0.0,
        )
        out_ref[...] = (
            jnp.dot(h, W2_ref[...], preferred_element_type=jnp.float32)
            + b2_ref[...]
        )

    return pl.pallas_call(
        mlp_body,
        grid=(B // BLK,),
        in_specs=[
            pl.BlockSpec((BLK, D), lambda i: (i, 0)),
            pl.BlockSpec((BLK, D), lambda i: (i, 0)),
            pl.BlockSpec((BLK, L), lambda i: (i, 0)),
            pl.BlockSpec((1, D), lambda i: (0, 0)),
            pl.BlockSpec((2 * D, H1), lambda i: (0, 0)),
            pl.BlockSpec((1, H1), lambda i: (0, 0)),
            pl.BlockSpec((H1, OUT), lambda i: (0, 0)),
            pl.BlockSpec((1, OUT), lambda i: (0, 0)),
        ],
        out_specs=pl.BlockSpec((BLK, OUT), lambda i: (i, 0)),
        out_shape=jax.ShapeDtypeStruct((B, OUT), jnp.float32),
        compiler_params=pltpu.CompilerParams(
            dimension_semantics=("parallel",),
        ),
    )(m, gsum, genre_ids, row0, W1, b1, W2, b2)


def kernel(movie_ids, genre_ids, movie_table, genre_table, W1, b1, W2, b2):
    mids = movie_ids.astype(jnp.int32)
    gids = genre_ids.astype(jnp.int32)
    gsum = _sc_genre_sum(gids.reshape(B * L), genre_table)
    mt_lin = _tc_pack(movie_table.T, 100352).reshape(100352, D)
    m = _sc_movie_rows(mids, mt_lin)
    row0 = genre_table[0:1, :]
    return _tc_mlp(m, gsum, gids, row0,
                   W1, b1.reshape(1, H1), W2, b2.reshape(1, OUT))


# speedup vs baseline: 1.4559x; 1.4348x over previous
import functools

import jax
import jax.numpy as jnp
from jax import lax
from jax.experimental import pallas as pl
from jax.experimental.pallas import tpu as pltpu
from jax.experimental.pallas import tpu_sc as plsc

B = 4096
L = 20
D = 32
H1 = 256
OUT = 64

NW = 32
BPW = B // NW
GPW = BPW * L
GCH = 128
NCH = GPW // GCH

_MESH = plsc.VectorSubcoreMesh(core_axis_name="c", subcore_axis_name="s")


def _sc_genre_sum(genre_ids_flat, genre_table):
    @functools.partial(
        pl.kernel,
        mesh=_MESH,
        out_type=jax.ShapeDtypeStruct((B, D), jnp.float32),
        scratch_types=[
            pltpu.VMEM((GPW,), jnp.int32),
            pltpu.VMEM((GPW, D), jnp.float32),
            pltpu.VMEM((BPW, D), jnp.float32),
            pltpu.SemaphoreType.DMA,
        ],
        compiler_params=pltpu.CompilerParams(use_tc_tiling_on_sc=False),
    )
    def genre_kernel(gids_hbm, gtab_hbm, out_hbm, gidx, grows, gsum, sem):
        wid = lax.axis_index("s") * 2 + lax.axis_index("c")
        base = wid * BPW
        pltpu.sync_copy(gids_hbm.at[pl.ds(base * L, GPW)], gidx)
        copies = []
        for j in range(NCH):
            copies.append(
                pltpu.async_copy(
                    gtab_hbm.at[gidx.at[pl.ds(j * GCH, GCH)]],
                    grows.at[pl.ds(j * GCH, GCH)],
                    sem,
                )
            )
        for cp in copies:
            cp.wait()

        def body(b, carry):
            for j in range(D // 16):
                acc = grows[b * L, pl.ds(j * 16, 16)]
                for l in range(1, L):
                    acc = acc + grows[b * L + l, pl.ds(j * 16, 16)]
                gsum[b, pl.ds(j * 16, 16)] = acc
            return carry

        lax.fori_loop(0, BPW, body, 0)
        pltpu.sync_copy(gsum, out_hbm.at[pl.ds(base, BPW)])

    return genre_kernel(genre_ids_flat, genre_table)


def _sc_movie_rows(movie_ids, movie_table):
    @functools.partial(
        pl.kernel,
        mesh=_MESH,
        out_type=jax.ShapeDtypeStruct((B, D), jnp.float32),
        scratch_types=[
            pltpu.VMEM((BPW,), jnp.int32),
            pltpu.VMEM((BPW, D), jnp.float32),
            pltpu.SemaphoreType.DMA,
        ],
        compiler_params=pltpu.CompilerParams(use_tc_tiling_on_sc=False),
    )
    def movie_kernel(mids_hbm, mtab_hbm, out_hbm, midx, mrows, sem):
        wid = lax.axis_index("s") * 2 + lax.axis_index("c")
        base = wid * BPW
        pltpu.sync_copy(mids_hbm.at[pl.ds(base, BPW)], midx)
        pltpu.async_copy(mtab_hbm.at[midx], mrows, sem).wait()
        pltpu.sync_copy(mrows, out_hbm.at[pl.ds(base, BPW)])

    return movie_kernel(movie_ids, movie_table)


def _tc_mlp(m, gsum, genre_ids, row0, W1, b1, W2, b2):
    BLK = 1024

    def mlp_body(m_ref, g_ref, gid_ref, row0_ref, W1_ref, b1_ref, W2_ref,
                 b2_ref, out_ref):
        gids = gid_ref[...]
        c0 = jnp.sum((gids == 0).astype(jnp.float32), axis=1, keepdims=True)
        denom = jnp.maximum(jnp.float32(L) - c0, 1.0)
        g = (g_ref[...] - c0 * row0_ref[...]) / denom
        h = jnp.maximum(
            jnp.dot(m_ref[...], W1_ref[:D, :], preferred_element_type=jnp.float32)
            + jnp.dot(g, W1_ref[D:, :], preferred_element_type=jnp.float32)
            + b1_ref[...],
            0.0,
        )
        out_ref[...] = (
            lax.dot_general(W2_ref[...], h, (((0,), (1,)), ((), ())),
                            preferred_element_type=jnp.float32)
            + b2_ref[...]
        )

    out_t = pl.pallas_call(
        mlp_body,
        grid=(B // BLK,),
        in_specs=[
            pl.BlockSpec((BLK, D), lambda i: (i, 0)),
            pl.BlockSpec((BLK, D), lambda i: (i, 0)),
            pl.BlockSpec((BLK, L), lambda i: (i, 0)),
            pl.BlockSpec((1, D), lambda i: (0, 0)),
            pl.BlockSpec((2 * D, H1), lambda i: (0, 0)),
            pl.BlockSpec((1, H1), lambda i: (0, 0)),
            pl.BlockSpec((H1, OUT), lambda i: (0, 0)),
            pl.BlockSpec((OUT, 1), lambda i: (0, 0)),
        ],
        out_specs=pl.BlockSpec((OUT, BLK), lambda i: (0, i)),
        out_shape=jax.ShapeDtypeStruct((OUT, B), jnp.float32),
        compiler_params=pltpu.CompilerParams(
            dimension_semantics=("parallel",),
        ),
    )(m, gsum, genre_ids, row0, W1, b1, W2, b2)
    return out_t.T


def kernel(movie_ids, genre_ids, movie_table, genre_table, W1, b1, W2, b2):
    mids = movie_ids.astype(jnp.int32)
    gids = genre_ids.astype(jnp.int32)
    gsum = _sc_genre_sum(gids.reshape(B * L), genre_table)
    m = _sc_movie_rows(mids, movie_table)
    row0 = genre_table[0:1, :]
    return _tc_mlp(m, gsum, gids, row0,
                   W1, b1.reshape(1, H1), W2, b2.reshape(OUT, 1))


# speedup vs baseline: 1.5451x; 1.0612x over previous
import functools

import jax
import jax.numpy as jnp
from jax import lax
from jax.experimental import pallas as pl
from jax.experimental.pallas import tpu as pltpu
from jax.experimental.pallas import tpu_sc as plsc

B = 4096
L = 20
D = 32
H1 = 256
OUT = 64

NW = 32
BPW = B // NW
GPW = BPW * L
GCH = 128
NCH = GPW // GCH

_MESH = plsc.VectorSubcoreMesh(core_axis_name="c", subcore_axis_name="s")


def _sc_genre_sum(genre_ids_flat, genre_table):
    @functools.partial(
        pl.kernel,
        mesh=_MESH,
        out_type=jax.ShapeDtypeStruct((B, 128), jnp.float32),
        scratch_types=[
            pltpu.VMEM((GPW,), jnp.int32),
            pltpu.VMEM((GPW, D), jnp.float32),
            pltpu.VMEM((BPW, D), jnp.float32),
            pltpu.SemaphoreType.DMA,
        ],
        compiler_params=pltpu.CompilerParams(use_tc_tiling_on_sc=False),
    )
    def genre_kernel(gids_hbm, gtab_hbm, out_hbm, gidx, grows, gsum, sem):
        wid = lax.axis_index("s") * 2 + lax.axis_index("c")
        base = wid * BPW
        pltpu.sync_copy(gids_hbm.at[pl.ds(base * L, GPW)], gidx)
        copies = []
        for j in range(NCH):
            copies.append(
                pltpu.async_copy(
                    gtab_hbm.at[gidx.at[pl.ds(j * GCH, GCH)]],
                    grows.at[pl.ds(j * GCH, GCH)],
                    sem,
                )
            )
        for cp in copies:
            cp.wait()

        def body(b, carry):
            for j in range(D // 16):
                acc = grows[b * L, pl.ds(j * 16, 16)]
                for l in range(1, L):
                    acc = acc + grows[b * L + l, pl.ds(j * 16, 16)]
                gsum[b, pl.ds(j * 16, 16)] = acc
            return carry

        lax.fori_loop(0, BPW, body, 0)
        pltpu.sync_copy(gsum, out_hbm.at[pl.ds(base, BPW), pl.ds(0, D)])

    return genre_kernel(genre_ids_flat, genre_table)


def _sc_movie_rows(movie_ids, movie_table):
    @functools.partial(
        pl.kernel,
        mesh=_MESH,
        out_type=jax.ShapeDtypeStruct((B, 128), jnp.float32),
        scratch_types=[
            pltpu.VMEM((BPW,), jnp.int32),
            pltpu.VMEM((BPW, D), jnp.float32),
            pltpu.SemaphoreType.DMA,
        ],
        compiler_params=pltpu.CompilerParams(use_tc_tiling_on_sc=False),
    )
    def movie_kernel(mids_hbm, mtab_hbm, out_hbm, midx, mrows, sem):
        wid = lax.axis_index("s") * 2 + lax.axis_index("c")
        base = wid * BPW
        pltpu.sync_copy(mids_hbm.at[pl.ds(base, BPW)], midx)
        pltpu.async_copy(mtab_hbm.at[midx], mrows, sem).wait()
        pltpu.sync_copy(mrows, out_hbm.at[pl.ds(base, BPW), pl.ds(0, D)])

    return movie_kernel(movie_ids, movie_table)


def _tc_mlp(m, gsum, genre_ids, row0, W1, b1, W2, b2):
    BLK = 1024

    def mlp_body(m_ref, g_ref, gid_ref, row0_ref, W1_ref, b1_ref, W2t_ref,
                 b2_ref, out_ref):
        gids = gid_ref[...]
        c0 = jnp.sum((gids == 0).astype(jnp.float32), axis=1, keepdims=True)
        denom = jnp.maximum(jnp.float32(L) - c0, 1.0)
        g = (g_ref[:, :D] - c0 * row0_ref[...]) / denom
        h = jnp.maximum(
            jnp.dot(m_ref[:, :D], W1_ref[:D, :],
                    preferred_element_type=jnp.float32)
            + jnp.dot(g, W1_ref[D:, :], preferred_element_type=jnp.float32)
            + b1_ref[...],
            0.0,
        )
        out_ref[...] = (
            lax.dot_general(W2t_ref[...], h, (((1,), (1,)), ((), ())),
                            preferred_element_type=jnp.float32)
            + b2_ref[...]
        )

    out_t = pl.pallas_call(
        mlp_body,
        grid=(B // BLK,),
        in_specs=[
            pl.BlockSpec((BLK, 128), lambda i: (i, 0)),
            pl.BlockSpec((BLK, 128), lambda i: (i, 0)),
            pl.BlockSpec((BLK, L), lambda i: (i, 0)),
            pl.BlockSpec((1, D), lambda i: (0, 0)),
            pl.BlockSpec((2 * D, H1), lambda i: (0, 0)),
            pl.BlockSpec((1, H1), lambda i: (0, 0)),
            pl.BlockSpec((OUT, H1), lambda i: (0, 0)),
            pl.BlockSpec((OUT, 1), lambda i: (0, 0)),
        ],
        out_specs=pl.BlockSpec((OUT, BLK), lambda i: (0, i)),
        out_shape=jax.ShapeDtypeStruct((OUT, B), jnp.float32),
        compiler_params=pltpu.CompilerParams(
            dimension_semantics=("parallel",),
        ),
    )(m, gsum, genre_ids, row0, W1, b1, W2, b2)
    return out_t.T


def kernel(movie_ids, genre_ids, movie_table, genre_table, W1, b1, W2, b2):
    mids = movie_ids.astype(jnp.int32)
    gids = genre_ids.astype(jnp.int32)
    gsum = _sc_genre_sum(gids.reshape(B * L), genre_table)
    m = _sc_movie_rows(mids, movie_table)
    row0 = genre_table[0:1, :]
    return _tc_mlp(m, gsum, gids, row0,
                   W1, b1.reshape(1, H1), W2.T, b2.reshape(OUT, 1))
